# Initial kernel scaffold; baseline (speedup 1.0000x reference)
import jax
import jax.numpy as jnp
from jax.experimental import pallas as pl


def kernel(x, weight, bias, gamma, beta):
    raise NotImplementedError("write your pallas kernel here")


# speedup vs baseline: 5.0610x; 5.0610x over previous
import functools

import jax
import jax.numpy as jnp
from jax import lax
from jax.experimental import pallas as pl
from jax.experimental.pallas import tpu as pltpu


def _conv_stats_kernel(xp_ref, w2_ref, b_ref, y_ref, stats_ref, p_ref, *,
                       H, W, Cin):
    HW = H * W
    col = lax.broadcasted_iota(jnp.int32, (1, HW), 1) % W
    not_first = col != 0
    not_last = col != (W - 1)

    for kh in range(3):
        for kw in range(3):
            dw = kw - 1
            t = kh * 3 + kw
            off = (kh + 1) * W + dw
            xs = xp_ref[:, off:off + HW]
            if dw == -1:
                xs = jnp.where(not_first, xs, 0.0)
            elif dw == 1:
                xs = jnp.where(not_last, xs, 0.0)
            p_ref[t * Cin:(t + 1) * Cin, :] = xs

    y = jnp.dot(w2_ref[...], p_ref[...], preferred_element_type=jnp.float32)
    y = y + b_ref[...]
    y_ref[...] = y
    s = jnp.sum(y, axis=1, keepdims=True)
    ss = jnp.sum(y * y, axis=1, keepdims=True)
    stats_ref[...] = jnp.concatenate([s, ss], axis=1)


def _bn_relu_kernel(y_ref, sc_ref, sh_ref, o_ref):
    o_ref[...] = jnp.maximum(y_ref[...] * sc_ref[...] + sh_ref[...], 0.0)


def kernel(x, weight, bias, gamma, beta, *, eps=1e-5):
    N, Cin, H, W = x.shape
    Cout = weight.shape[0]
    HW = H * W
    HpW = (H + 4) * W

    xp = jnp.pad(x, ((0, 0), (0, 0), (2, 2), (0, 0))).reshape(N, Cin, HpW)
    w2 = jnp.transpose(weight, (0, 2, 3, 1)).reshape(Cout, 9 * Cin)
    b2 = bias.reshape(Cout, 1)

    vmem_limit = 64 * 1024 * 1024

    y, stats = pl.pallas_call(
        functools.partial(_conv_stats_kernel, H=H, W=W, Cin=Cin),
        grid=(N,),
        in_specs=[
            pl.BlockSpec((None, Cin, HpW), lambda n: (n, 0, 0)),
            pl.BlockSpec((Cout, 9 * Cin), lambda n: (0, 0)),
            pl.BlockSpec((Cout, 1), lambda n: (0, 0)),
        ],
        out_specs=(
            pl.BlockSpec((None, Cout, HW), lambda n: (n, 0, 0)),
            pl.BlockSpec((None, Cout, 2), lambda n: (n, 0, 0)),
        ),
        out_shape=(
            jax.ShapeDtypeStruct((N, Cout, HW), jnp.float32),
            jax.ShapeDtypeStruct((N, Cout, 2), jnp.float32),
        ),
        scratch_shapes=[pltpu.VMEM((9 * Cin, HW), jnp.float32)],
        compiler_params=pltpu.CompilerParams(
            dimension_semantics=("parallel",),
            vmem_limit_bytes=vmem_limit),
    )(xp, w2, b2)

    count = jnp.float32(N * H * W)
    tot = jnp.sum(stats, axis=0)
    mean = tot[:, 0] / count
    var = jnp.maximum(tot[:, 1] / count - mean * mean, 0.0)
    inv = lax.rsqrt(var + eps)
    scale = (gamma * inv).reshape(Cout, 1)
    shift = (beta - mean * gamma * inv).reshape(Cout, 1)

    out = pl.pallas_call(
        _bn_relu_kernel,
        grid=(N,),
        in_specs=[
            pl.BlockSpec((None, Cout, HW), lambda n: (n, 0, 0)),
            pl.BlockSpec((Cout, 1), lambda n: (0, 0)),
            pl.BlockSpec((Cout, 1), lambda n: (0, 0)),
        ],
        out_specs=pl.BlockSpec((None, Cout, HW), lambda n: (n, 0, 0)),
        out_shape=jax.ShapeDtypeStruct((N, Cout, HW), jnp.float32),
        compiler_params=pltpu.CompilerParams(
            dimension_semantics=("parallel",),
            vmem_limit_bytes=vmem_limit),
    )(y, scale, shift)

    return out.reshape(N, Cout, H, W)


# speedup vs baseline: 5.4251x; 1.0719x over previous
import functools

import jax
import jax.numpy as jnp
from jax import lax
from jax.experimental import pallas as pl
from jax.experimental.pallas import tpu as pltpu


def _conv_stats_kernel(x_ref, w2_ref, b_ref, y_ref, stats_ref, p_ref, *,
                       H, W, Cin):
    HW = H * W
    lane = lax.broadcasted_iota(jnp.int32, (1, HW), 1)
    col = lane % W

    x = x_ref[...]
    xm = jnp.where(col != (W - 1), x, 0.0)
    xp = jnp.where(col != 0, x, 0.0)

    for kh in range(3):
        dh = kh - 1
        if dh == -1:
            rmask = lane >= W
        elif dh == 1:
            rmask = lane < (HW - W)
        else:
            rmask = None
        for kw in range(3):
            dw = kw - 1
            t = kh * 3 + kw
            src = xm if dw == -1 else (xp if dw == 1 else x)
            delta = dh * W + dw
            shifted = pltpu.roll(src, (-delta) % HW, axis=1)
            if rmask is not None:
                shifted = jnp.where(rmask, shifted, 0.0)
            p_ref[t * Cin:(t + 1) * Cin, :] = shifted

    y = jnp.dot(w2_ref[...], p_ref[...], preferred_element_type=jnp.float32)
    y = y + b_ref[...]
    s = jnp.sum(y, axis=1, keepdims=True)
    ss = jnp.sum(y * y, axis=1, keepdims=True)
    stats_ref[...] = jnp.concatenate([s, ss], axis=1)
    y_ref[...] = y.astype(jnp.bfloat16)


def _bn_relu_kernel(y_ref, sc_ref, sh_ref, o_ref):
    y = y_ref[...].astype(jnp.float32)
    o_ref[...] = jnp.maximum(y * sc_ref[...] + sh_ref[...], 0.0)


def kernel(x, weight, bias, gamma, beta, *, eps=1e-5):
    N, Cin, H, W = x.shape
    Cout = weight.shape[0]
    HW = H * W

    xf = x.reshape(N, Cin, HW)
    w2 = jnp.transpose(weight, (0, 2, 3, 1)).reshape(Cout, 9 * Cin)
    b2 = bias.reshape(Cout, 1)

    vmem_limit = 64 * 1024 * 1024

    y, stats = pl.pallas_call(
        functools.partial(_conv_stats_kernel, H=H, W=W, Cin=Cin),
        grid=(N,),
        in_specs=[
            pl.BlockSpec((None, Cin, HW), lambda n: (n, 0, 0)),
            pl.BlockSpec((Cout, 9 * Cin), lambda n: (0, 0)),
            pl.BlockSpec((Cout, 1), lambda n: (0, 0)),
        ],
        out_specs=(
            pl.BlockSpec((None, Cout, HW), lambda n: (n, 0, 0)),
            pl.BlockSpec((None, Cout, 2), lambda n: (n, 0, 0)),
        ),
        out_shape=(
            jax.ShapeDtypeStruct((N, Cout, HW), jnp.bfloat16),
            jax.ShapeDtypeStruct((N, Cout, 2), jnp.float32),
        ),
        scratch_shapes=[pltpu.VMEM((9 * Cin, HW), jnp.float32)],
        compiler_params=pltpu.CompilerParams(
            dimension_semantics=("parallel",),
            vmem_limit_bytes=vmem_limit),
    )(xf, w2, b2)

    count = jnp.float32(N * H * W)
    tot = jnp.sum(stats, axis=0)
    mean = tot[:, 0] / count
    var = jnp.maximum(tot[:, 1] / count - mean * mean, 0.0)
    inv = lax.rsqrt(var + eps)
    scale = (gamma * inv).reshape(Cout, 1)
    shift = (beta - mean * gamma * inv).reshape(Cout, 1)

    out = pl.pallas_call(
        _bn_relu_kernel,
        grid=(N,),
        in_specs=[
            pl.BlockSpec((None, Cout, HW), lambda n: (n, 0, 0)),
            pl.BlockSpec((Cout, 1), lambda n: (0, 0)),
            pl.BlockSpec((Cout, 1), lambda n: (0, 0)),
        ],
        out_specs=pl.BlockSpec((None, Cout, HW), lambda n: (n, 0, 0)),
        out_shape=jax.ShapeDtypeStruct((N, Cout, HW), jnp.float32),
        compiler_params=pltpu.CompilerParams(
            dimension_semantics=("parallel",),
            vmem_limit_bytes=vmem_limit),
    )(y, scale, shift)

    return out.reshape(N, Cout, H, W)


# speedup vs baseline: 6.7079x; 1.2364x over previous
import functools

import jax
import jax.numpy as jnp
from jax import lax
from jax.experimental import pallas as pl
from jax.experimental.pallas import tpu as pltpu


def _conv_stats_kernel(x_ref, w2_ref, y_ref, stats_ref, p_ref, *,
                       H, W, Cin, IMG):
    HW = H * W
    lane = lax.broadcasted_iota(jnp.int32, (1, HW), 1)
    col = lane % W
    row_lo = lane >= W
    row_hi = lane < (HW - W)
    not_last = col != (W - 1)
    not_first = col != 0

    s_acc = jnp.zeros((w2_ref.shape[0], 1), jnp.float32)
    ss_acc = jnp.zeros((w2_ref.shape[0], 1), jnp.float32)
    for i in range(IMG):
        x = x_ref[i]
        xm = jnp.where(not_last, x, 0.0)
        xp = jnp.where(not_first, x, 0.0)
        for kh in range(3):
            dh = kh - 1
            rmask = row_lo if dh == -1 else (row_hi if dh == 1 else None)
            for kw in range(3):
                dw = kw - 1
                t = kh * 3 + kw
                src = xm if dw == -1 else (xp if dw == 1 else x)
                delta = dh * W + dw
                shifted = pltpu.roll(src, (-delta) % HW, axis=1)
                if rmask is not None:
                    shifted = jnp.where(rmask, shifted, 0.0)
                p_ref[t * Cin:(t + 1) * Cin, :] = shifted.astype(jnp.bfloat16)

        y = jnp.dot(w2_ref[...], p_ref[...],
                    preferred_element_type=jnp.float32)
        s_acc += jnp.sum(y, axis=1, keepdims=True)
        ss_acc += jnp.sum(y * y, axis=1, keepdims=True)
        y_ref[i] = y.astype(jnp.bfloat16)
    stats_ref[...] = jnp.concatenate([s_acc, ss_acc], axis=1)


def _bn_relu_kernel(y_ref, sc_ref, sh_ref, o_ref, *, IMG):
    for i in range(IMG):
        y = y_ref[i].astype(jnp.float32)
        o_ref[i] = jnp.maximum(y * sc_ref[...] + sh_ref[...], 0.0)


def kernel(x, weight, bias, gamma, beta, *, eps=1e-5):
    N, Cin, H, W = x.shape
    Cout = weight.shape[0]
    HW = H * W
    IMG = 4 if N % 4 == 0 else (2 if N % 2 == 0 else 1)
    NB = N // IMG

    xf = x.reshape(N, Cin, HW)
    w2 = jnp.transpose(weight, (0, 2, 3, 1)).reshape(Cout, 9 * Cin)
    w2 = w2.astype(jnp.bfloat16)

    vmem_limit = 100 * 1024 * 1024

    y, stats = pl.pallas_call(
        functools.partial(_conv_stats_kernel, H=H, W=W, Cin=Cin, IMG=IMG),
        grid=(NB,),
        in_specs=[
            pl.BlockSpec((IMG, Cin, HW), lambda n: (n, 0, 0)),
            pl.BlockSpec((Cout, 9 * Cin), lambda n: (0, 0)),
        ],
        out_specs=(
            pl.BlockSpec((IMG, Cout, HW), lambda n: (n, 0, 0)),
            pl.BlockSpec((None, Cout, 2), lambda n: (n, 0, 0)),
        ),
        out_shape=(
            jax.ShapeDtypeStruct((N, Cout, HW), jnp.bfloat16),
            jax.ShapeDtypeStruct((NB, Cout, 2), jnp.float32),
        ),
        scratch_shapes=[pltpu.VMEM((9 * Cin, HW), jnp.bfloat16)],
        compiler_params=pltpu.CompilerParams(
            dimension_semantics=("parallel",),
            vmem_limit_bytes=vmem_limit),
    )(xf, w2)

    count = jnp.float32(N * H * W)
    tot = jnp.sum(stats, axis=0)
    mean = tot[:, 0] / count
    var = jnp.maximum(tot[:, 1] / count - mean * mean, 0.0)
    inv = lax.rsqrt(var + eps)
    scale = (gamma * inv).reshape(Cout, 1)
    shift = (beta - mean * gamma * inv).reshape(Cout, 1)

    out = pl.pallas_call(
        functools.partial(_bn_relu_kernel, IMG=IMG),
        grid=(NB,),
        in_specs=[
            pl.BlockSpec((IMG, Cout, HW), lambda n: (n, 0, 0)),
            pl.BlockSpec((Cout, 1), lambda n: (0, 0)),
            pl.BlockSpec((Cout, 1), lambda n: (0, 0)),
        ],
        out_specs=pl.BlockSpec((IMG, Cout, HW), lambda n: (n, 0, 0)),
        out_shape=jax.ShapeDtypeStruct((N, Cout, HW), jnp.float32),
        compiler_params=pltpu.CompilerParams(
            dimension_semantics=("parallel",),
            vmem_limit_bytes=vmem_limit),
    )(y, scale, shift)

    return out.reshape(N, Cout, H, W)
